# baseline (speedup 1.0000x reference)
import functools

import jax
import jax.numpy as jnp
from jax import lax
from jax.experimental import pallas as pl
from jax.experimental.pallas import tpu as pltpu
from jax.experimental.pallas import tpu_sc as plsc

N = 10000
E = 320000
D_IN = 128
D_HID = 16
D_OUT = 40

NC, NS = 2, 16
CH = 128
IDX_BLK = 8
NPAD = 10240
ROWS_PER_TILE = NPAD // NS
EROWS = E // CH


_SC_MESH = plsc.VectorSubcoreMesh(core_axis_name="c", subcore_axis_name="s")
_SC_PARAMS = pltpu.CompilerParams(use_tc_tiling_on_sc=False)


@functools.partial(
    pl.kernel,
    out_type=jax.ShapeDtypeStruct((NC, NPAD), jnp.float32),
    mesh=_SC_MESH,
    compiler_params=_SC_PARAMS,
    scratch_types=[
        pltpu.VMEM((IDX_BLK, CH), jnp.int32),
        pltpu.VMEM((CH,), jnp.float32),
        pltpu.VMEM_SHARED((NPAD,), jnp.float32),
    ],
)
def _sc_degree(e3_hbm, zero_hbm, out_hbm, didx_v, ones_v, acc_sh):
    c = lax.axis_index("c")
    s = lax.axis_index("s")
    w = c * NS + s
    r0 = s * ROWS_PER_TILE
    for i in range(CH // 16):
        ones_v[pl.ds(i * 16, 16)] = jnp.ones((16,), jnp.float32)
    pltpu.sync_copy(zero_hbm.at[pl.ds(r0, ROWS_PER_TILE)],
                    acc_sh.at[pl.ds(r0, ROWS_PER_TILE)])
    plsc.subcore_barrier()

    base = w * 78

    def blk(i, carry):
        row0 = base + i * IDX_BLK
        pltpu.sync_copy(e3_hbm.at[1, pl.ds(row0, IDX_BLK)], didx_v)
        for j in range(IDX_BLK):
            pltpu.sync_copy(ones_v, acc_sh.at[didx_v.at[j]], add=True)
        return carry

    lax.fori_loop(0, 9, blk, 0)
    pltpu.sync_copy(e3_hbm.at[1, pl.ds(base + 72, 6)],
                    didx_v.at[pl.ds(0, 6)])
    for j in range(6):
        pltpu.sync_copy(ones_v, acc_sh.at[didx_v.at[j]], add=True)

    @pl.when(w < 4)
    def _extra():
        pltpu.sync_copy(e3_hbm.at[1, pl.ds(2496 + w, 1)],
                        didx_v.at[pl.ds(0, 1)])
        pltpu.sync_copy(ones_v, acc_sh.at[didx_v.at[0]], add=True)

    plsc.subcore_barrier()
    pltpu.sync_copy(acc_sh.at[pl.ds(r0, ROWS_PER_TILE)],
                    out_hbm.at[c, pl.ds(r0, ROWS_PER_TILE)])


def _make_sc_agg(depth):
    @functools.partial(
        pl.kernel,
        out_type=jax.ShapeDtypeStruct((NC, NPAD, depth), jnp.float32),
        mesh=_SC_MESH,
        compiler_params=_SC_PARAMS,
        scratch_types=(
            [pltpu.VMEM((IDX_BLK, CH), jnp.int32) for _ in range(4)]
            + [pltpu.VMEM((CH, depth), jnp.float32)
               for _ in range(2 * IDX_BLK)]
            + [pltpu.VMEM_SHARED((NPAD, depth), jnp.float32)]
            + [pltpu.SemaphoreType.DMA for _ in range(2 * IDX_BLK + 4)]
        ),
    )
    def agg(g_hbm, e3_hbm, zero_hbm, out_hbm, *scr):
        sidx = scr[0:2]
        didx = scr[2:4]
        rows = scr[4:4 + 2 * IDX_BLK]
        acc_sh = scr[4 + 2 * IDX_BLK]
        gsem = scr[5 + 2 * IDX_BLK:5 + 4 * IDX_BLK]
        ssem = scr[5 + 4 * IDX_BLK:]
        c = lax.axis_index("c")
        s = lax.axis_index("s")
        r0 = s * ROWS_PER_TILE
        grp = IDX_BLK // 2
        pltpu.sync_copy(zero_hbm.at[pl.ds(r0, ROWS_PER_TILE)],
                        acc_sh.at[pl.ds(r0, ROWS_PER_TILE)])
        plsc.subcore_barrier()

        def drain_scat(g):
            for _ in range(grp):
                pltpu.make_async_copy(
                    g_hbm.at[pl.ds(0, CH)], rows[0], ssem[g]).wait()

        def run_block(i, row0, p):
            pltpu.sync_copy(e3_hbm.at[0, pl.ds(row0, IDX_BLK)], sidx[p])
            pltpu.sync_copy(e3_hbm.at[1, pl.ds(row0, IDX_BLK)], didx[p])
            for g in range(2):
                gg = p * 2 + g
                @pl.when(i >= 2)
                def _(gg=gg):
                    drain_scat(gg)
                cps = []
                for j in range(grp):
                    k = g * grp + j
                    sk = p * IDX_BLK + k
                    cps.append(pltpu.async_copy(
                        g_hbm.at[sidx[p].at[k]], rows[sk], gsem[sk]))
                for j in range(grp):
                    k = g * grp + j
                    sk = p * IDX_BLK + k
                    cps[j].wait()
                    pltpu.async_copy(rows[sk], acc_sh.at[didx[p].at[k]],
                                     ssem[gg], add=True)

        def pipelined(base, nblk):
            def blk(i, carry):
                row0 = base + i * IDX_BLK
                par = lax.rem(i, 2)
                for p in range(2):
                    @pl.when(par == p)
                    def _(p=p):
                        run_block(i, row0, p)
                return carry

            lax.fori_loop(0, nblk, blk, 0)
            for gg in range(4):
                drain_scat(gg)

        def tail(row0, count):
            pltpu.sync_copy(e3_hbm.at[0, pl.ds(row0, count)],
                            sidx[0].at[pl.ds(0, count)])
            pltpu.sync_copy(e3_hbm.at[1, pl.ds(row0, count)],
                            didx[0].at[pl.ds(0, count)])
            for j in range(count):
                pltpu.sync_copy(g_hbm.at[sidx[0].at[j]], rows[0])
                pltpu.sync_copy(rows[0], acc_sh.at[didx[0].at[j]], add=True)

        w = c * NS + s
        pipelined(w * 78, 9)
        tail(w * 78 + 72, 6)

        @pl.when(w < 4)
        def _extra():
            tail(2496 + w, 1)

        plsc.subcore_barrier()
        pltpu.sync_copy(acc_sh.at[pl.ds(r0, ROWS_PER_TILE)],
                        out_hbm.at[c, pl.ds(r0, ROWS_PER_TILE)])

    return agg


_sc_agg_hid = _make_sc_agg(D_HID)
_sc_agg_out = _make_sc_agg(D_OUT)


def _tc0_body(x_ref, w_ref, h_ref):
    h_ref[...] = jnp.dot(x_ref[...], w_ref[...],
                         preferred_element_type=jnp.float32)


_tc0 = pl.pallas_call(
    _tc0_body,
    out_shape=jax.ShapeDtypeStruct((N, D_HID), jnp.float32),
)


def _tc1_body(h_ref, cnt_ref, g_ref, dinv_ref):
    csum = cnt_ref[:, 0:1] + cnt_ref[:, 1:2]
    dinv = jax.lax.rsqrt(csum + 1.0)
    dinv_ref[...] = dinv
    g_ref[...] = h_ref[...] * dinv


_tc1 = pl.pallas_call(
    _tc1_body,
    out_shape=[
        jax.ShapeDtypeStruct((N, D_HID), jnp.float32),
        jax.ShapeDtypeStruct((N, 1), jnp.float32),
    ],
)


def _tc2_body(agg_ref, g1_ref, dinv_ref, b1_ref, w2_ref, g2_ref):
    agg = agg_ref[0, :N] + agg_ref[1, :N] + g1_ref[...]
    dinv = dinv_ref[...]
    h1 = jnp.maximum(agg * dinv + b1_ref[...], 0.0)
    g2_ref[...] = jnp.dot(h1, w2_ref[...],
                          preferred_element_type=jnp.float32) * dinv


_tc2 = pl.pallas_call(
    _tc2_body,
    out_shape=jax.ShapeDtypeStruct((N, D_OUT), jnp.float32),
)


def _tc3_body(agg_ref, g2_ref, dinv_ref, b2_ref, o_ref):
    o = (agg_ref[0, :N] + agg_ref[1, :N] + g2_ref[...]) * dinv_ref[...] \
        + b2_ref[...]
    m = jnp.max(o, axis=1, keepdims=True)
    lse = m + jnp.log(jnp.sum(jnp.exp(o - m), axis=1, keepdims=True))
    o_ref[...] = o - lse


_tc3 = pl.pallas_call(
    _tc3_body,
    out_shape=jax.ShapeDtypeStruct((N, D_OUT), jnp.float32),
)


def kernel(x, edge_index, W1, b1, W2, b2):
    e3 = edge_index.reshape(2, EROWS, CH)

    cnt = _sc_degree(e3, jnp.zeros((NPAD,), jnp.float32))
    h = _tc0(x, W1)
    g1, dinv = _tc1(h, cnt.T[:N])

    agg1 = _sc_agg_hid(g1, e3, jnp.zeros((NPAD, D_HID), jnp.float32))
    g2 = _tc2(agg1, g1, dinv, b1.reshape(1, D_HID), W2)

    agg2 = _sc_agg_out(g2, e3, jnp.zeros((NPAD, D_OUT), jnp.float32))
    return _tc3(agg2, g2, dinv, b2.reshape(1, D_OUT))
